# baseline (device time: 21995 ns/iter reference)
import jax
import jax.numpy as jnp
from jax import lax
from jax.experimental import pallas as pl
from jax.experimental.pallas import tpu as pltpu

N_DEV = 16
WIN = 128


def kernel(x, Wq, K_ext, V_ext, Wo):
    B, Sq, Dm = x.shape
    _, Skv, Hq, Dh = K_ext.shape
    D = Hq * Dh
    Se = Skv + 2 * WIN

    def body(x_ref, wq_ref, k_ref, v_ref, wo_ref, out_ref,
             kbuf, vbuf, send_sems, recv_sems):
        me = lax.axis_index("i")
        left = lax.rem(me - 1 + N_DEV, N_DEV)
        right = lax.rem(me + 1, N_DEV)

        barrier_sem = pltpu.get_barrier_semaphore()
        for nbr in (left, right):
            pl.semaphore_signal(barrier_sem, inc=1, device_id=(nbr,),
                                device_id_type=pl.DeviceIdType.MESH)
        pl.semaphore_wait(barrier_sem, 2)

        kbuf[:, WIN:WIN + Skv, :] = k_ref[...].reshape(B, Skv, D)
        vbuf[:, WIN:WIN + Skv, :] = v_ref[...].reshape(B, Skv, D)

        plan = [
            (kbuf, Skv, 0, right, 0),
            (kbuf, WIN, WIN + Skv, left, 1),
            (vbuf, Skv, 0, right, 2),
            (vbuf, WIN, WIN + Skv, left, 3),
        ]
        rdmas = []
        for buf, src_row, dst_row, tgt, i in plan:
            r = pltpu.make_async_remote_copy(
                src_ref=buf.at[:, pl.ds(src_row, WIN), :],
                dst_ref=buf.at[:, pl.ds(dst_row, WIN), :],
                send_sem=send_sems.at[i],
                recv_sem=recv_sems.at[i],
                device_id=(tgt,),
                device_id_type=pl.DeviceIdType.MESH,
            )
            r.start()
            rdmas.append(r)

        qproj = [
            jnp.dot(x_ref[b], wq_ref[...], preferred_element_type=jnp.float32)
            for b in range(B)
        ]

        for r in rdmas:
            r.wait()

        qi = lax.broadcasted_iota(jnp.int32, (Sq, Se), 0)
        kj = lax.broadcasted_iota(jnp.int32, (Sq, Se), 1)
        diff = kj - qi
        kg = me * Skv - WIN + kj
        mask = (diff >= 0) & (diff <= 2 * WIN) & (kg >= 0) & (kg < N_DEV * Skv)

        for b in range(B):
            kb = kbuf[b]
            vb = vbuf[b]
            ctxs = []
            for h in range(Hq):
                q = qproj[b][:, h * Dh:(h + 1) * Dh]
                kh = kb[:, h * Dh:(h + 1) * Dh]
                vh = vb[:, h * Dh:(h + 1) * Dh]
                s = lax.dot_general(
                    q, kh, (((1,), (1,)), ((), ())),
                    preferred_element_type=jnp.float32,
                ) * 0.125
                s = jnp.where(mask, s, jnp.float32(-1e9))
                m = jnp.max(s, axis=-1, keepdims=True)
                w = jnp.exp(s - m)
                w = w / jnp.sum(w, axis=-1, keepdims=True)
                ctxs.append(jnp.dot(w, vh, preferred_element_type=jnp.float32))
            ctx = jnp.concatenate(ctxs, axis=1)
            out_ref[b] = jnp.dot(ctx, wo_ref[...],
                                 preferred_element_type=jnp.float32)

    return pl.pallas_call(
        body,
        out_shape=jax.ShapeDtypeStruct((B, Sq, Dm), jnp.float32),
        in_specs=[pl.BlockSpec(memory_space=pltpu.VMEM)] * 5,
        out_specs=pl.BlockSpec(memory_space=pltpu.VMEM),
        scratch_shapes=[
            pltpu.VMEM((B, Se, D), jnp.float32),
            pltpu.VMEM((B, Se, D), jnp.float32),
            pltpu.SemaphoreType.DMA((4,)),
            pltpu.SemaphoreType.DMA((4,)),
        ],
        compiler_params=pltpu.CompilerParams(collective_id=0),
    )(x, Wq, K_ext, V_ext, Wo)


# device time: 19225 ns/iter; 1.1441x vs baseline; 1.1441x over previous
import jax
import jax.numpy as jnp
from jax import lax
from jax.experimental import pallas as pl
from jax.experimental.pallas import tpu as pltpu

N_DEV = 16
WIN = 128


def kernel(x, Wq, K_ext, V_ext, Wo):
    B, Sq, Dm = x.shape
    _, Skv, Hq, Dh = K_ext.shape
    D = Hq * Dh
    Se = Skv + 2 * WIN

    def body(x_ref, wq_ref, k_ref, v_ref, wo_ref, out_ref,
             kbuf, vbuf, send_sems, recv_sems):
        me = lax.axis_index("i")
        left = lax.rem(me - 1 + N_DEV, N_DEV)
        right = lax.rem(me + 1, N_DEV)

        barrier_sem = pltpu.get_barrier_semaphore()
        for nbr in (left, right):
            pl.semaphore_signal(barrier_sem, inc=1, device_id=(nbr,),
                                device_id_type=pl.DeviceIdType.MESH)
        pl.semaphore_wait(barrier_sem, 2)

        kbuf[:, WIN:WIN + Skv, :] = k_ref[...].reshape(B, Skv, D).astype(jnp.bfloat16)
        vbuf[:, WIN:WIN + Skv, :] = v_ref[...].reshape(B, Skv, D).astype(jnp.bfloat16)

        plan = [
            (kbuf, Skv, 0, right, 0),
            (kbuf, WIN, WIN + Skv, left, 1),
            (vbuf, Skv, 0, right, 2),
            (vbuf, WIN, WIN + Skv, left, 3),
        ]
        rdmas = []
        for buf, src_row, dst_row, tgt, i in plan:
            r = pltpu.make_async_remote_copy(
                src_ref=buf.at[:, pl.ds(src_row, WIN), :],
                dst_ref=buf.at[:, pl.ds(dst_row, WIN), :],
                send_sem=send_sems.at[i],
                recv_sem=recv_sems.at[i],
                device_id=(tgt,),
                device_id_type=pl.DeviceIdType.MESH,
            )
            r.start()
            rdmas.append(r)

        wq16 = wq_ref[...].astype(jnp.bfloat16)
        qproj = [
            jnp.dot(x_ref[b].astype(jnp.bfloat16), wq16,
                    preferred_element_type=jnp.float32).astype(jnp.bfloat16)
            for b in range(B)
        ]

        for r in rdmas:
            r.wait()

        qi = lax.broadcasted_iota(jnp.int32, (Sq, Se), 0)
        kj = lax.broadcasted_iota(jnp.int32, (Sq, Se), 1)
        diff = kj - qi
        kg = me * Skv - WIN + kj
        mask = (diff >= 0) & (diff <= 2 * WIN) & (kg >= 0) & (kg < N_DEV * Skv)

        for b in range(B):
            kb = kbuf[b]
            vb = vbuf[b]
            ctxs = []
            for h in range(Hq):
                q = qproj[b][:, h * Dh:(h + 1) * Dh]
                kh = kb[:, h * Dh:(h + 1) * Dh]
                vh = vb[:, h * Dh:(h + 1) * Dh]
                s = lax.dot_general(
                    q, kh, (((1,), (1,)), ((), ())),
                    preferred_element_type=jnp.float32,
                ) * 0.125
                s = jnp.where(mask, s, jnp.float32(-1e9))
                m = jnp.max(s, axis=-1, keepdims=True)
                w = jnp.exp(s - m)
                w = (w / jnp.sum(w, axis=-1, keepdims=True)).astype(jnp.bfloat16)
                ctxs.append(jnp.dot(w, vh, preferred_element_type=jnp.float32))
            ctx = jnp.concatenate(ctxs, axis=1).astype(jnp.bfloat16)
            out_ref[b] = jnp.dot(ctx, wo_ref[...].astype(jnp.bfloat16),
                                 preferred_element_type=jnp.float32)

    return pl.pallas_call(
        body,
        out_shape=jax.ShapeDtypeStruct((B, Sq, Dm), jnp.float32),
        in_specs=[pl.BlockSpec(memory_space=pltpu.VMEM)] * 5,
        out_specs=pl.BlockSpec(memory_space=pltpu.VMEM),
        scratch_shapes=[
            pltpu.VMEM((B, Se, D), jnp.bfloat16),
            pltpu.VMEM((B, Se, D), jnp.bfloat16),
            pltpu.SemaphoreType.DMA((4,)),
            pltpu.SemaphoreType.DMA((4,)),
        ],
        compiler_params=pltpu.CompilerParams(collective_id=0),
    )(x, Wq, K_ext, V_ext, Wo)


# device time: 18235 ns/iter; 1.2062x vs baseline; 1.0543x over previous
import jax
import jax.numpy as jnp
from jax import lax
from jax.experimental import pallas as pl
from jax.experimental.pallas import tpu as pltpu

N_DEV = 16
WIN = 128


def kernel(x, Wq, K_ext, V_ext, Wo):
    B, Sq, Dm = x.shape
    _, Skv, Hq, Dh = K_ext.shape
    D = Hq * Dh
    Se = Skv + 2 * WIN

    def body(x_ref, wq_ref, k_ref, v_ref, wo_ref, out_ref,
             kbuf, vbuf, send_sems, recv_sems):
        me = lax.axis_index("i")
        left = lax.rem(me - 1 + N_DEV, N_DEV)
        right = lax.rem(me + 1, N_DEV)

        barrier_sem = pltpu.get_barrier_semaphore()
        for nbr in (left, right):
            pl.semaphore_signal(barrier_sem, inc=1, device_id=(nbr,),
                                device_id_type=pl.DeviceIdType.MESH)
        pl.semaphore_wait(barrier_sem, 2)

        kbuf[:, WIN:WIN + Skv, :] = k_ref[...].reshape(B, Skv, D).astype(jnp.bfloat16)
        vbuf[:, WIN:WIN + Skv, :] = v_ref[...].reshape(B, Skv, D).astype(jnp.bfloat16)

        plan = [
            (kbuf, Skv, 0, right, 0),
            (kbuf, WIN, WIN + Skv, left, 1),
            (vbuf, Skv, 0, right, 2),
            (vbuf, WIN, WIN + Skv, left, 3),
        ]
        rdmas = []
        for buf, src_row, dst_row, tgt, i in plan:
            r = pltpu.make_async_remote_copy(
                src_ref=buf.at[:, pl.ds(src_row, WIN), :],
                dst_ref=buf.at[:, pl.ds(dst_row, WIN), :],
                send_sem=send_sems.at[i],
                recv_sem=recv_sems.at[i],
                device_id=(tgt,),
                device_id_type=pl.DeviceIdType.MESH,
            )
            r.start()
            rdmas.append(r)

        wq16 = (wq_ref[...] * 0.125).astype(jnp.bfloat16)
        qall = jnp.dot(
            x_ref[...].reshape(B * Sq, Dm).astype(jnp.bfloat16), wq16,
            preferred_element_type=jnp.float32,
        ).astype(jnp.bfloat16)

        KB = 3 * WIN
        biases = []
        for qb in range(Sq // WIN):
            qi = lax.broadcasted_iota(jnp.int32, (WIN, KB), 0) + qb * WIN
            kj = lax.broadcasted_iota(jnp.int32, (WIN, KB), 1) + qb * WIN
            diff = kj - qi
            kg = me * Skv - WIN + kj
            mask = (diff >= 0) & (diff <= 2 * WIN) & (kg >= 0) & (kg < N_DEV * Skv)
            biases.append(jnp.where(mask, 0.0, -1e9).astype(jnp.float32))

        for r in rdmas:
            r.wait()

        ctx_rows = []
        for b in range(B):
            kb = kbuf[b]
            vb = vbuf[b]
            for qb in range(Sq // WIN):
                koff = qb * WIN
                ctxs = []
                for h in range(Hq):
                    q = qall[b * Sq + koff:b * Sq + koff + WIN,
                             h * Dh:(h + 1) * Dh]
                    kh = kb[koff:koff + KB, h * Dh:(h + 1) * Dh]
                    vh = vb[koff:koff + KB, h * Dh:(h + 1) * Dh]
                    s = lax.dot_general(
                        q, kh, (((1,), (1,)), ((), ())),
                        preferred_element_type=jnp.float32,
                    ) + biases[qb]
                    w = jnp.exp(s)
                    wsum = jnp.sum(w, axis=-1, keepdims=True)
                    c = jnp.dot(w.astype(jnp.bfloat16), vh,
                                preferred_element_type=jnp.float32)
                    ctxs.append(c / wsum)
                ctx_rows.append(jnp.concatenate(ctxs, axis=1))
        ctx = jnp.concatenate(ctx_rows, axis=0).astype(jnp.bfloat16)
        out = jnp.dot(ctx, wo_ref[...].astype(jnp.bfloat16),
                      preferred_element_type=jnp.float32)
        out_ref[...] = out.reshape(B, Sq, Dm)

    return pl.pallas_call(
        body,
        out_shape=jax.ShapeDtypeStruct((B, Sq, Dm), jnp.float32),
        in_specs=[pl.BlockSpec(memory_space=pltpu.VMEM)] * 5,
        out_specs=pl.BlockSpec(memory_space=pltpu.VMEM),
        scratch_shapes=[
            pltpu.VMEM((B, Se, D), jnp.bfloat16),
            pltpu.VMEM((B, Se, D), jnp.bfloat16),
            pltpu.SemaphoreType.DMA((4,)),
            pltpu.SemaphoreType.DMA((4,)),
        ],
        compiler_params=pltpu.CompilerParams(collective_id=0),
    )(x, Wq, K_ext, V_ext, Wo)
